# baseline (speedup 1.0000x reference)
import functools

import jax
import jax.numpy as jnp
from jax import lax
from jax.experimental import pallas as pl
from jax.experimental.pallas import tpu as pltpu
from jax.experimental.pallas import tpu_sc as plsc

BATCH = 64
XDIM = 3072
D = 256
L = 16

_NW = 8
_RPW = BATCH // _NW
_LN2 = 0.6931471805599453


def _select_body(noise_ref, u_ref, tgt_ref, jflat_ref, tsel_ref):
    noise = noise_ref[...]
    u = u_ref[...]
    r = jnp.clip(jnp.floor(u * jnp.float32(XDIM)).astype(jnp.int32), 0, XDIM - 1)

    lo = jnp.zeros((BATCH, 1), jnp.int32)
    hi = jnp.full((BATCH, 1), XDIM - 1, jnp.int32)
    ghi = jnp.zeros((BATCH, 1), jnp.int32)
    for _ in range(12):
        mid = (lo + hi + 1) >> 1
        cnt = jnp.sum((noise >= mid).astype(jnp.int32), axis=1, keepdims=True)
        ok = cnt >= r + 1
        lo = jnp.where(ok, mid, lo)
        hi = jnp.where(ok, hi, mid - 1)
        ghi = jnp.where(ok, ghi, cnt)
    v = lo

    m = r - ghi
    colidx = jax.lax.broadcasted_iota(jnp.int32, (BATCH, XDIM), 1)
    eqcol = jnp.where(noise == v, colidx, XDIM)

    lo2 = jnp.zeros((BATCH, 1), jnp.int32)
    hi2 = jnp.full((BATCH, 1), XDIM - 1, jnp.int32)
    for _ in range(12):
        mid = (lo2 + hi2) >> 1
        cnt = jnp.sum((eqcol <= mid).astype(jnp.int32), axis=1, keepdims=True)
        ok = cnt >= m + 1
        lo2 = jnp.where(ok, lo2, mid + 1)
        hi2 = jnp.where(ok, mid, hi2)
    j = lo2

    b_iota = jax.lax.broadcasted_iota(jnp.int32, (BATCH, 1), 0)
    jflat_ref[...] = b_iota * XDIM + j
    tsel_ref[...] = jnp.sum(jnp.where(colidx == j, tgt_ref[...], 0), axis=1,
                            keepdims=True)


def _vlog(s_vec):
    bits = lax.bitcast_convert_type(s_vec, jnp.int32)
    e = ((bits >> 23) & 255) - 127
    frac = (bits & 0x7FFFFF).astype(jnp.float32) * jnp.float32(1.0 / 8388608.0)
    y = (e.astype(jnp.float32) + frac) * jnp.float32(_LN2)
    for _ in range(3):
        y = y + s_vec * jnp.exp(-y) - 1.0
    return y


_DNUMS = lax.GatherDimensionNumbers(
    offset_dims=(), collapsed_slice_dims=(0,), start_index_map=(0,))


def _shuffle(x, idx):
    return lax.gather(x, idx[:, None], _DNUMS, slice_sizes=(1,),
                      mode=lax.GatherScatterMode.PROMISE_IN_BOUNDS)


def _vreduce(x, op):
    lane = lax.iota(jnp.int32, L)
    for k in (8, 4, 2, 1):
        x = op(x, _shuffle(x, jnp.bitwise_xor(lane, k)))
    return x


def _sc_body(logits_hbm, jf_hbm, ts_hbm, part_hbm, out_hbm,
             idx_v, tsr_v, rows_v, acc_v, out_v, red_v, sem):
    wid = lax.axis_index("s")

    @pl.when(wid < _NW)
    def _():
        base = wid * _RPW
        pltpu.sync_copy(jf_hbm.at[pl.ds(base, _RPW)], idx_v)
        pltpu.sync_copy(ts_hbm.at[pl.ds(base, _RPW)], tsr_v.at[pl.ds(0, _RPW)])
        pltpu.async_copy(logits_hbm.at[idx_v], rows_v, sem).wait()
        lane = lax.iota(jnp.int32, L)
        tv16 = tsr_v[...]
        acc = jnp.zeros((L,), jnp.float32)
        for rloc in range(_RPW):
            mx = rows_v[rloc, pl.ds(0, L)]
            for c in range(1, D // L):
                mx = jnp.maximum(mx, rows_v[rloc, pl.ds(c * L, L)])
            m_vec = _vreduce(mx, jnp.maximum)
            t_vec = _shuffle(tv16, jnp.full((L,), rloc, jnp.int32))
            sv = jnp.zeros((L,), jnp.float32)
            pk = jnp.zeros((L,), jnp.float32)
            for c in range(D // L):
                chunk = rows_v[rloc, pl.ds(c * L, L)]
                sv = sv + jnp.exp(chunk - m_vec)
                pk = pk + jnp.where(lane + (c * L) == t_vec, chunk, 0.0)
            s_vec = _vreduce(sv, jnp.add)
            picked = _vreduce(pk, jnp.add)
            acc = acc + picked - m_vec - _vlog(s_vec)
        acc_v[...] = acc
        pltpu.sync_copy(acc_v, part_hbm.at[wid])

    plsc.subcore_barrier()

    @pl.when(wid == 0)
    def _():
        pltpu.sync_copy(part_hbm, red_v)
        tot = red_v[0, :]
        for w in range(1, _NW):
            tot = tot + red_v[w, :]
        out_v[...] = tot * jnp.float32(1.0 / BATCH)
        pltpu.sync_copy(out_v, out_hbm)


@jax.jit
def kernel(x, logits, rand_noise, u, targets):
    del x
    tgt_flat = targets.reshape(BATCH, XDIM)
    u2 = u.reshape(BATCH, 1)

    jflat, tsel = pl.pallas_call(
        _select_body,
        out_shape=(
            jax.ShapeDtypeStruct((BATCH, 1), jnp.int32),
            jax.ShapeDtypeStruct((BATCH, 1), jnp.int32),
        ),
    )(rand_noise, u2, tgt_flat)

    logits2d = logits.reshape(BATCH * XDIM, D)
    sc_fused = functools.partial(
        pl.kernel,
        out_type=(jax.ShapeDtypeStruct((_NW, L), jnp.float32),
                  jax.ShapeDtypeStruct((L,), jnp.float32)),
        mesh=plsc.VectorSubcoreMesh(core_axis_name="c", subcore_axis_name="s",
                                    num_cores=1),
        scratch_types=[
            pltpu.VMEM((_RPW,), jnp.int32),
            pltpu.VMEM((L,), jnp.int32),
            pltpu.VMEM((_RPW, D), jnp.float32),
            pltpu.VMEM((L,), jnp.float32),
            pltpu.VMEM((L,), jnp.float32),
            pltpu.VMEM((_NW, L), jnp.float32),
            pltpu.SemaphoreType.DMA,
        ],
    )(_sc_body)
    _, out = sc_fused(logits2d, jflat.reshape(BATCH), tsel.reshape(BATCH))
    return out[:1].reshape(())
